# baseline (speedup 1.0000x reference)
import jax
import jax.numpy as jnp
from jax import lax
from jax.experimental import pallas as pl
from jax.experimental.pallas import tpu as pltpu
from jax.experimental.pallas import tpu_sc as plsc

_N = 1048576
_C = 89
_NC = 2
_NS = 16
_NW = _NC * _NS
_RPW = _N // _NW
_CHUNK = 512
_NCHUNK = _RPW // _CHUNK
_G = _CHUNK // 16
_MAP_PAD = 96
_NT = _MAP_PAD // 16


def _sc_body(a_hbm, map_hbm, out_hbm, map_v, a_v, oh_v, cols_v):
    wid = lax.axis_index("s") * _NC + lax.axis_index("c")
    pltpu.sync_copy(map_hbm, map_v)

    tabs = [jnp.maximum(map_v[pl.ds(16 * k, 16)], 0) for k in range(_NT)]

    lane = lax.iota(jnp.int32, 16)
    ones16 = jnp.ones((16,), jnp.float32)
    zeros16 = jnp.zeros((16,), jnp.float32)

    def zero_row(r, carry):
        rr = jnp.broadcast_to(r, (16,)).astype(jnp.int32)
        for k in range(_NT):
            cc = jnp.minimum(k * 16 + lane, _C - 1)
            plsc.store_scatter(oh_v, [rr, cc], zeros16)
        return carry

    lax.fori_loop(0, _CHUNK, zero_row, 0)

    def lookup(av):
        av = jnp.clip(av, 0, _C)
        sub = av & 15
        hi = av >> 4
        idx = tabs[0].at[sub].get(mode="promise_in_bounds")
        for k in range(1, _NT):
            t = tabs[k].at[sub].get(mode="promise_in_bounds")
            idx = jnp.where(hi == k, t, idx)
        return idx

    row_base = wid * _RPW

    def chunk_step(ci, carry):
        base = row_base + ci * _CHUNK
        pltpu.sync_copy(a_hbm.at[pl.ds(base, _CHUNK)], a_v)

        def mark(g, c):
            av = a_v[pl.ds(g * 16, 16)]
            idx = lookup(av)
            rows = g * 16 + lane
            cols_v[pl.ds(g * 16, 16)] = idx
            plsc.store_scatter(oh_v, [rows, idx], ones16)
            return c

        lax.fori_loop(0, _G, mark, 0)
        pltpu.sync_copy(oh_v, out_hbm.at[pl.ds(base, _CHUNK)])

        def clear(g, c):
            idx = cols_v[pl.ds(g * 16, 16)]
            rows = g * 16 + lane
            plsc.store_scatter(oh_v, [rows, idx], zeros16)
            return c

        lax.fori_loop(0, _G, clear, 0)
        return carry

    lax.fori_loop(0, _NCHUNK, chunk_step, 0)


def kernel(atomic_numbers, index_map):
    a = atomic_numbers.astype(jnp.int32)
    m = jnp.pad(index_map.astype(jnp.int32), (0, _MAP_PAD - index_map.shape[0]))
    mesh = plsc.VectorSubcoreMesh(core_axis_name="c", subcore_axis_name="s")
    return pl.kernel(
        _sc_body,
        out_type=jax.ShapeDtypeStruct((_N, _C), jnp.float32),
        mesh=mesh,
        compiler_params=pltpu.CompilerParams(needs_layout_passes=False),
        scratch_types=[
            pltpu.VMEM((_MAP_PAD,), jnp.int32),
            pltpu.VMEM((_CHUNK,), jnp.int32),
            pltpu.VMEM((_CHUNK, _C), jnp.float32),
            pltpu.VMEM((_CHUNK,), jnp.int32),
        ],
    )(a, m)
